# baseline (device time: 17833 ns/iter reference)
import jax
import jax.numpy as jnp
from jax import lax
from jax.experimental import pallas as pl
from jax.experimental.pallas import tpu as pltpu

N_DEV = 4


def kernel(x, w_mat):
    m_per, k = x.shape
    _, n_per = w_mat.shape
    m_half = m_per // 2

    def body(x_ref, w_ref, out_ref,
             x_vmem, w_vmem, out_vmem, mine, from_l, from_r, diag_a, diag_b,
             send_sems, recv_sems, load_sems, store_sems):
        my_pos = lax.axis_index("i")
        left = lax.rem(my_pos + N_DEV - 1, N_DEV)
        right = lax.rem(my_pos + 1, N_DEV)
        diag = lax.rem(my_pos + 2, N_DEV)

        x_load0 = pltpu.make_async_copy(
            x_ref.at[pl.ds(0, m_half), :], x_vmem.at[0], load_sems.at[0])
        x_load1 = pltpu.make_async_copy(
            x_ref.at[pl.ds(m_half, m_half), :], x_vmem.at[1], load_sems.at[1])
        w_load = pltpu.make_async_copy(w_ref, w_vmem, load_sems.at[2])
        x_load0.start()
        x_load1.start()
        w_load.start()

        barrier_sem = pltpu.get_barrier_semaphore()
        for nbr in [left, right]:
            pl.semaphore_signal(
                barrier_sem, inc=1,
                device_id=(nbr,), device_id_type=pl.DeviceIdType.MESH,
            )

        x_load0.wait()
        mine[0] = x_vmem[0].astype(jnp.bfloat16)
        x_load1.wait()
        mine[1] = x_vmem[1].astype(jnp.bfloat16)
        pl.semaphore_wait(barrier_sem, 2)

        def rcopy(src, dst, sem_idx, dev):
            return pltpu.make_async_remote_copy(
                src_ref=src, dst_ref=dst,
                send_sem=send_sems.at[sem_idx], recv_sem=recv_sems.at[sem_idx],
                device_id=(dev,), device_id_type=pl.DeviceIdType.MESH,
            )

        sends = [
            rcopy(mine.at[0], from_l.at[0], 0, right),
            rcopy(mine.at[1], from_r.at[1], 1, left),
            rcopy(mine.at[1], from_l.at[1], 2, right),
            rcopy(mine.at[0], from_r.at[0], 3, left),
        ]
        for s in sends:
            s.start()

        out_stores = []

        def gemm_store(src_block, origin_row, blk):
            out_vmem[pl.ds(blk * m_half, m_half), :] = jnp.dot(
                src_block, w, preferred_element_type=jnp.float32
            ).astype(jnp.bfloat16)
            st = pltpu.make_async_copy(
                out_vmem.at[pl.ds(blk * m_half, m_half), :],
                out_ref.at[pl.ds(origin_row, m_half), :],
                store_sems.at[blk])
            st.start()
            out_stores.append(st)

        w_load.wait()
        w = w_vmem[...].astype(jnp.bfloat16)
        gemm_store(mine[0], my_pos * m_per, 0)
        gemm_store(mine[1], my_pos * m_per + m_half, 1)

        rcopy(from_l.at[0], from_l.at[0], 0, left).wait_recv()
        fwd_r = rcopy(from_l.at[0], diag_a, 4, right)
        fwd_r.start()
        rcopy(from_r.at[1], from_r.at[1], 1, right).wait_recv()
        fwd_l = rcopy(from_r.at[1], diag_b, 5, left)
        fwd_l.start()

        rcopy(from_l.at[1], from_l.at[1], 2, left).wait_recv()
        gemm_store(from_l[0], left * m_per, 2)
        gemm_store(from_l[1], left * m_per + m_half, 3)

        rcopy(from_r.at[0], from_r.at[0], 3, right).wait_recv()
        gemm_store(from_r[0], right * m_per, 4)
        gemm_store(from_r[1], right * m_per + m_half, 5)

        rcopy(diag_a, diag_a, 4, left).wait_recv()
        gemm_store(diag_a[...], diag * m_per, 6)
        rcopy(diag_b, diag_b, 5, right).wait_recv()
        gemm_store(diag_b[...], diag * m_per + m_half, 7)

        for s in sends:
            s.wait_send()
        fwd_r.wait_send()
        fwd_l.wait_send()
        for st in out_stores:
            st.wait()

    out_shape = jax.ShapeDtypeStruct((N_DEV * m_per, n_per), jnp.bfloat16)
    return pl.pallas_call(
        body,
        out_shape=out_shape,
        in_specs=[
            pl.BlockSpec(memory_space=pltpu.MemorySpace.HBM),
            pl.BlockSpec(memory_space=pltpu.MemorySpace.HBM),
        ],
        out_specs=pl.BlockSpec(memory_space=pltpu.MemorySpace.HBM),
        scratch_shapes=[
            pltpu.VMEM((2, m_half, k), jnp.float32),
            pltpu.VMEM((k, n_per), jnp.float32),
            pltpu.VMEM((N_DEV * m_per, n_per), jnp.bfloat16),
            pltpu.VMEM((2, m_half, k), jnp.bfloat16),
            pltpu.VMEM((2, m_half, k), jnp.bfloat16),
            pltpu.VMEM((2, m_half, k), jnp.bfloat16),
            pltpu.VMEM((m_half, k), jnp.bfloat16),
            pltpu.VMEM((m_half, k), jnp.bfloat16),
            pltpu.SemaphoreType.DMA((6,)),
            pltpu.SemaphoreType.DMA((6,)),
            pltpu.SemaphoreType.DMA((3,)),
            pltpu.SemaphoreType.DMA((8,)),
        ],
        compiler_params=pltpu.CompilerParams(collective_id=0),
    )(x, w_mat)


# device time: 17831 ns/iter; 1.0001x vs baseline; 1.0001x over previous
import jax
import jax.numpy as jnp
from jax import lax
from jax.experimental import pallas as pl
from jax.experimental.pallas import tpu as pltpu

N_DEV = 4


def kernel(x, w_mat):
    m_per, k = x.shape
    _, n_per = w_mat.shape
    m_half = m_per // 2

    def body(x_ref, w_ref, out_ref,
             out_vmem, mine, from_l, from_r, diag_a, diag_b,
             send_sems, recv_sems, store_sems):
        my_pos = lax.axis_index("i")
        left = lax.rem(my_pos + N_DEV - 1, N_DEV)
        right = lax.rem(my_pos + 1, N_DEV)
        diag = lax.rem(my_pos + 2, N_DEV)

        barrier_sem = pltpu.get_barrier_semaphore()
        for nbr in [left, right]:
            pl.semaphore_signal(
                barrier_sem, inc=1,
                device_id=(nbr,), device_id_type=pl.DeviceIdType.MESH,
            )

        mine[0] = x_ref[:m_half, :].astype(jnp.bfloat16)
        mine[1] = x_ref[m_half:, :].astype(jnp.bfloat16)
        pl.semaphore_wait(barrier_sem, 2)

        def rcopy(src, dst, sem_idx, dev):
            return pltpu.make_async_remote_copy(
                src_ref=src, dst_ref=dst,
                send_sem=send_sems.at[sem_idx], recv_sem=recv_sems.at[sem_idx],
                device_id=(dev,), device_id_type=pl.DeviceIdType.MESH,
            )

        sends = [
            rcopy(mine.at[0], from_l.at[0], 0, right),
            rcopy(mine.at[1], from_r.at[1], 1, left),
            rcopy(mine.at[1], from_l.at[1], 2, right),
            rcopy(mine.at[0], from_r.at[0], 3, left),
        ]
        for s in sends:
            s.start()

        out_stores = []

        def gemm_store(src_block, origin_row, blk, rows=m_half):
            off = blk * m_half if blk < 6 else 6 * m_half + (blk - 6) * (
                m_half // 2)
            out_vmem[pl.ds(off, rows), :] = jnp.dot(
                src_block, w, preferred_element_type=jnp.float32
            ).astype(jnp.bfloat16)
            st = pltpu.make_async_copy(
                out_vmem.at[pl.ds(off, rows), :],
                out_ref.at[pl.ds(origin_row, rows), :],
                store_sems.at[blk])
            st.start()
            out_stores.append(st)

        w = w_ref[...].astype(jnp.bfloat16)
        gemm_store(mine[0], my_pos * m_per, 0)
        gemm_store(mine[1], my_pos * m_per + m_half, 1)

        m_q = m_half // 2
        rcopy(from_l.at[0], from_l.at[0], 0, left).wait_recv()
        fwds = [
            rcopy(from_l.at[0, pl.ds(0, m_q), :],
                  diag_a.at[pl.ds(0, m_q), :], 4, right),
            rcopy(from_l.at[0, pl.ds(m_q, m_q), :],
                  diag_a.at[pl.ds(m_q, m_q), :], 5, right),
        ]
        fwds[0].start()
        fwds[1].start()
        rcopy(from_r.at[1], from_r.at[1], 1, right).wait_recv()
        fwds += [
            rcopy(from_r.at[1, pl.ds(0, m_q), :],
                  diag_b.at[pl.ds(0, m_q), :], 6, left),
            rcopy(from_r.at[1, pl.ds(m_q, m_q), :],
                  diag_b.at[pl.ds(m_q, m_q), :], 7, left),
        ]
        fwds[2].start()
        fwds[3].start()

        rcopy(from_l.at[1], from_l.at[1], 2, left).wait_recv()
        gemm_store(from_l[0], left * m_per, 2)
        gemm_store(from_l[1], left * m_per + m_half, 3)

        rcopy(from_r.at[0], from_r.at[0], 3, right).wait_recv()
        gemm_store(from_r[0], right * m_per, 4)
        gemm_store(from_r[1], right * m_per + m_half, 5)

        rcopy(diag_a.at[pl.ds(0, m_q), :], diag_a.at[pl.ds(0, m_q), :],
              4, left).wait_recv()
        gemm_store(diag_a[:m_q, :], diag * m_per, 6, m_q)
        rcopy(diag_a.at[pl.ds(m_q, m_q), :], diag_a.at[pl.ds(m_q, m_q), :],
              5, left).wait_recv()
        gemm_store(diag_a[m_q:, :], diag * m_per + m_q, 7, m_q)
        rcopy(diag_b.at[pl.ds(0, m_q), :], diag_b.at[pl.ds(0, m_q), :],
              6, right).wait_recv()
        gemm_store(diag_b[:m_q, :], diag * m_per + m_half, 8, m_q)
        rcopy(diag_b.at[pl.ds(m_q, m_q), :], diag_b.at[pl.ds(m_q, m_q), :],
              7, right).wait_recv()
        gemm_store(diag_b[m_q:, :], diag * m_per + m_half + m_q, 9, m_q)

        for s in sends:
            s.wait_send()
        for f in fwds:
            f.wait_send()
        for st in out_stores:
            st.wait()

    out_shape = jax.ShapeDtypeStruct((N_DEV * m_per, n_per), jnp.bfloat16)
    return pl.pallas_call(
        body,
        out_shape=out_shape,
        in_specs=[
            pl.BlockSpec(memory_space=pltpu.VMEM),
            pl.BlockSpec(memory_space=pltpu.VMEM),
        ],
        out_specs=pl.BlockSpec(memory_space=pltpu.MemorySpace.HBM),
        scratch_shapes=[
            pltpu.VMEM((N_DEV * m_per, n_per), jnp.bfloat16),
            pltpu.VMEM((2, m_half, k), jnp.bfloat16),
            pltpu.VMEM((2, m_half, k), jnp.bfloat16),
            pltpu.VMEM((2, m_half, k), jnp.bfloat16),
            pltpu.VMEM((m_half, k), jnp.bfloat16),
            pltpu.VMEM((m_half, k), jnp.bfloat16),
            pltpu.SemaphoreType.DMA((8,)),
            pltpu.SemaphoreType.DMA((8,)),
            pltpu.SemaphoreType.DMA((10,)),
        ],
        compiler_params=pltpu.CompilerParams(collective_id=0),
    )(x, w_mat)


# device time: 17586 ns/iter; 1.0140x vs baseline; 1.0139x over previous
import jax
import jax.numpy as jnp
from jax import lax
from jax.experimental import pallas as pl
from jax.experimental.pallas import tpu as pltpu

N_DEV = 4


def kernel(x, w_mat):
    m_per, k = x.shape
    _, n_per = w_mat.shape
    m_half = m_per // 2

    def body(x_ref, w_ref, out_ref,
             out_vmem, mine, from_l, from_r, diag_a, diag_b,
             send_sems, recv_sems, store_sems):
        my_pos = lax.axis_index("i")
        left = lax.rem(my_pos + N_DEV - 1, N_DEV)
        right = lax.rem(my_pos + 1, N_DEV)
        diag = lax.rem(my_pos + 2, N_DEV)

        barrier_sem = pltpu.get_barrier_semaphore()
        for nbr in [left, right]:
            pl.semaphore_signal(
                barrier_sem, inc=1,
                device_id=(nbr,), device_id_type=pl.DeviceIdType.MESH,
            )

        mine[0] = x_ref[:m_half, :].astype(jnp.bfloat16)
        mine[1] = x_ref[m_half:, :].astype(jnp.bfloat16)
        pl.semaphore_wait(barrier_sem, 2)

        def rcopy(src, dst, sem_idx, dev):
            return pltpu.make_async_remote_copy(
                src_ref=src, dst_ref=dst,
                send_sem=send_sems.at[sem_idx], recv_sem=recv_sems.at[sem_idx],
                device_id=(dev,), device_id_type=pl.DeviceIdType.MESH,
            )

        sends = [
            rcopy(mine.at[0], from_l.at[0], 0, right),
            rcopy(mine.at[1], from_r.at[1], 1, left),
            rcopy(mine.at[1], from_l.at[1], 2, right),
            rcopy(mine.at[0], from_r.at[0], 3, left),
        ]
        for s in sends:
            s.start()

        out_stores = []

        def gemm_store(src_block, origin_row, blk):
            out_vmem[pl.ds(blk * m_half, m_half), :] = jnp.dot(
                src_block, w, preferred_element_type=jnp.float32
            ).astype(jnp.bfloat16)
            st = pltpu.make_async_copy(
                out_vmem.at[pl.ds(blk * m_half, m_half), :],
                out_ref.at[pl.ds(origin_row, m_half), :],
                store_sems.at[blk])
            st.start()
            out_stores.append(st)

        w = w_ref[...].astype(jnp.bfloat16)
        gemm_store(mine[0], my_pos * m_per, 0)
        gemm_store(mine[1], my_pos * m_per + m_half, 1)

        rcopy(from_l.at[0], from_l.at[0], 0, left).wait_recv()
        fwd_r = rcopy(from_l.at[0], diag_a, 4, right)
        fwd_r.start()
        rcopy(from_r.at[1], from_r.at[1], 1, right).wait_recv()
        fwd_l = rcopy(from_r.at[1], diag_b, 5, left)
        fwd_l.start()

        rcopy(from_l.at[1], from_l.at[1], 2, left).wait_recv()
        gemm_store(from_l[0], left * m_per, 2)
        gemm_store(from_l[1], left * m_per + m_half, 3)

        rcopy(from_r.at[0], from_r.at[0], 3, right).wait_recv()
        gemm_store(from_r[0], right * m_per, 4)
        gemm_store(from_r[1], right * m_per + m_half, 5)

        rcopy(diag_a, diag_a, 4, left).wait_recv()
        gemm_store(diag_a[...], diag * m_per, 6)
        rcopy(diag_b, diag_b, 5, right).wait_recv()
        gemm_store(diag_b[...], diag * m_per + m_half, 7)

        for s in sends:
            s.wait_send()
        fwd_r.wait_send()
        fwd_l.wait_send()
        for st in out_stores:
            st.wait()

    out_shape = jax.ShapeDtypeStruct((N_DEV * m_per, n_per), jnp.bfloat16)
    return pl.pallas_call(
        body,
        out_shape=out_shape,
        in_specs=[
            pl.BlockSpec(memory_space=pltpu.VMEM),
            pl.BlockSpec(memory_space=pltpu.VMEM),
        ],
        out_specs=pl.BlockSpec(memory_space=pltpu.MemorySpace.HBM),
        scratch_shapes=[
            pltpu.VMEM((N_DEV * m_per, n_per), jnp.bfloat16),
            pltpu.VMEM((2, m_half, k), jnp.bfloat16),
            pltpu.VMEM((2, m_half, k), jnp.bfloat16),
            pltpu.VMEM((2, m_half, k), jnp.bfloat16),
            pltpu.VMEM((m_half, k), jnp.bfloat16),
            pltpu.VMEM((m_half, k), jnp.bfloat16),
            pltpu.SemaphoreType.DMA((6,)),
            pltpu.SemaphoreType.DMA((6,)),
            pltpu.SemaphoreType.DMA((8,)),
        ],
        compiler_params=pltpu.CompilerParams(collective_id=0),
    )(x, w_mat)
